# baseline (speedup 1.0000x reference)
import jax
import jax.numpy as jnp
from jax.experimental import pallas as pl

THR = 0.5
N, D, H, W = 4, 48, 48, 48
LANES = W * 6
VC = (H + 1) * (W + 1) * 3

def _off(dz, dy, dx):
    return (dz * (H + 1) + dy) * (W + 1) + dx

_QUADS = [
    [(0, 0, 0), (0, 0, 1), (0, 1, 0), (0, 1, 1)],
    [(1, 0, 0), (1, 0, 1), (1, 1, 0), (1, 1, 1)],
    [(1, 0, 0), (1, 0, 1), (0, 0, 0), (0, 0, 1)],
    [(0, 1, 0), (0, 1, 1), (1, 1, 0), (1, 1, 1)],
    [(1, 0, 0), (0, 0, 0), (1, 1, 0), (0, 1, 0)],
    [(0, 0, 1), (1, 0, 1), (0, 1, 1), (1, 1, 1)],
]
OFFS = [[_off(*q[i]) for i in (0, 1, 2, 1, 2, 3)] for q in _QUADS]
ROLLS = [(0, 1), (0, -1), (1, 1), (1, -1), (2, 1), (2, -1)]


def _cubify_kernel(vox_ref, faces_ref, verts_ref):
    d = pl.program_id(1)
    vox = vox_ref[0]
    occ = vox > THR

    lane = jax.lax.broadcasted_iota(jnp.int32, (W, LANES), 1)
    row = jax.lax.broadcasted_iota(jnp.int32, (W, LANES), 0)
    expand = (lane // 6 == row).astype(jnp.float32)

    z = jax.lax.broadcasted_iota(jnp.int32, (D, H, LANES), 0)
    y = jax.lax.broadcasted_iota(jnp.int32, (D, H, LANES), 1)
    l = jax.lax.broadcasted_iota(jnp.int32, (D, H, LANES), 2)
    base = (z * (H + 1) + y) * (W + 1) + l // 6
    u = l % 6

    def branch(di):
        def f():
            axis, amt = ROLLS[di]
            nbr = jnp.roll(vox, amt, axis=axis) > THR
            mask = (occ & (~nbr)).astype(jnp.float32)
            maske = jax.lax.dot_general(
                mask.reshape(D * H, W), expand, (((1,), (0,)), ((), ())),
                preferred_element_type=jnp.float32).reshape(D, H, LANES)
            offs = OFFS[di]
            val = base + offs[0]
            for k in range(1, 6):
                val = jnp.where(u == k, base + offs[k], val)
            return jnp.where(maske > 0.5, val, -1)
        return f

    faces_ref[0, 0] = jax.lax.switch(d, [branch(i) for i in range(6)])

    @pl.when(d == 0)
    def _():
        i = jax.lax.broadcasted_iota(jnp.int32, (D + 1, VC), 0)
        c = jax.lax.broadcasted_iota(jnp.int32, (D + 1, VC), 1)
        w = c % 3
        jk = c // 3
        coord = jnp.where(w == 0, i,
                          jnp.where(w == 1, jk // (W + 1), jk % (W + 1)))
        verts_ref[0] = coord.astype(jnp.float32) - 0.5


@jax.jit
def kernel(voxel_probas):
    faces5, verts3 = pl.pallas_call(
        _cubify_kernel,
        grid=(N, 6),
        in_specs=[pl.BlockSpec((1, D, H, W), lambda n, d: (n, 0, 0, 0))],
        out_specs=[
            pl.BlockSpec((1, 1, D, H, LANES), lambda n, d: (n, d, 0, 0, 0)),
            pl.BlockSpec((1, D + 1, VC), lambda n, d: (n, 0, 0)),
        ],
        out_shape=[
            jax.ShapeDtypeStruct((N, 6, D, H, LANES), jnp.int32),
            jax.ShapeDtypeStruct((N, D + 1, VC), jnp.float32),
        ],
    )(voxel_probas)
    faces = faces5.reshape(N, 6 * D * H * W * 2, 3)
    verts = verts3.reshape(N, (D + 1) * (H + 1) * (W + 1), 3)
    return verts, faces


# speedup vs baseline: 1.1022x; 1.1022x over previous
import jax
import jax.numpy as jnp
from jax.experimental import pallas as pl
from jax.experimental.pallas import tpu as pltpu

THR = 0.5
N, D, H, W = 4, 48, 48, 48
NF = 6 * D * H * W * 2
FB = NF // 128
ROWS = FB * N
CH = 18
RPC = ROWS // CH
VC = (H + 1) * (W + 1) * 3

def _off(dz, dy, dx):
    return (dz * (H + 1) + dy) * (W + 1) + dx

_QUADS = [
    [(0, 0, 0), (0, 0, 1), (0, 1, 0), (0, 1, 1)],
    [(1, 0, 0), (1, 0, 1), (1, 1, 0), (1, 1, 1)],
    [(1, 0, 0), (1, 0, 1), (0, 0, 0), (0, 0, 1)],
    [(0, 1, 0), (0, 1, 1), (1, 1, 0), (1, 1, 1)],
    [(1, 0, 0), (0, 0, 0), (1, 1, 0), (0, 1, 0)],
    [(0, 0, 1), (1, 0, 1), (0, 1, 1), (1, 1, 1)],
]
OFFS = [[_off(*q[i]) for i in (0, 1, 2, 1, 2, 3)] for q in _QUADS]

ROLLS = [(0, 1), (0, -1), (1, 1), (1, -1), (2, 1), (2, -1)]


def _mask_kernel(vox_ref, m_ref):
    c = pl.program_id(0)
    n = pl.program_id(1)
    for d in range(6):
        @pl.when(c == d)
        def _(d=d):
            axis, amt = ROLLS[d]
            v = vox_ref[pl.ds(n, 1)][0]
            nbr = jnp.roll(v, amt, axis=axis)
            m = jnp.where((v > THR) & (nbr <= THR), 1.0, 0.0)
            m = jnp.repeat(m, 2, axis=2).reshape(D, H * 2 * W)
            m_ref[0, 0] = m.astype(jnp.bfloat16)


def _faces_kernel(m_ref, faces_ref, verts_ref, bscr_ref, vscr_ref):
    c = pl.program_id(0)
    w = pl.program_id(1)

    @pl.when(jnp.logical_and(c == 0, w == 0))
    def _():
        s = jax.lax.broadcasted_iota(jnp.int32, (RPC, 128), 0)
        l = jax.lax.broadcasted_iota(jnp.int32, (RPC, 128), 1)
        g = (s // 4) * 128 + l
        t = g % 2
        h = g // 2
        x = h % W
        y = (h // W) % H
        z = h // (W * H)
        bscr_ref[...] = ((z * (H + 1) + y) * (W + 1) + x) * 2 + t
        i = jax.lax.broadcasted_iota(jnp.int32, (D + 1, VC), 0)
        cc = jax.lax.broadcasted_iota(jnp.int32, (D + 1, VC), 1)
        wc = cc % 3
        jk = cc // 3
        coord = jnp.where(wc == 0, i,
                          jnp.where(wc == 1, jk // (W + 1), jk % (W + 1)))
        vscr_ref[...] = coord.astype(jnp.float32) - 0.5

    q = c % 3
    d = c // 3
    parts = [
        m_ref[0, pl.ds(n * D + q * 16, 16), :].reshape(RPC // 4, 128)
        for n in range(N)]
    mblk = jnp.stack(parts, axis=1).reshape(RPC, 128)

    def mk_branch(k):
        def f():
            o0 = jnp.where(w == 0, OFFS[k][0],
                           jnp.where(w == 1, OFFS[k][1], OFFS[k][2]))
            o1 = jnp.where(w == 0, OFFS[k][3],
                           jnp.where(w == 1, OFFS[k][4], OFFS[k][5]))
            return jnp.stack([o0, o1])
        return f

    ot = jax.lax.switch(d, [mk_branch(k) for k in range(6)])
    bt = bscr_ref[...]
    base = bt >> 1
    off = jnp.where((bt & 1) == 0, ot[0], ot[1])
    val = base + (off + q * 16 * (H + 1) * (W + 1))
    faces_ref[0] = jnp.where(mblk > 0.5, val, -1)

    @pl.when(jnp.logical_and(c % 5 == 0, w == 0))
    def _():
        verts_ref[0] = vscr_ref[...]


@jax.jit
def kernel(voxel_probas):
    masks = pl.pallas_call(
        _mask_kernel,
        grid=(6, N),
        in_specs=[pl.BlockSpec((N, D, H, W), lambda c, n: (0, 0, 0, 0))],
        out_specs=pl.BlockSpec((1, 1, D, H * 2 * W), lambda c, n: (c, n, 0, 0)),
        out_shape=jax.ShapeDtypeStruct((6, N, D, H * 2 * W), jnp.bfloat16),
    )(voxel_probas)
    masks = masks.reshape(6, N * D, H * 2 * W)
    faces3, verts3 = pl.pallas_call(
        _faces_kernel,
        grid=(CH, 3),
        in_specs=[pl.BlockSpec((1, N * D, H * 2 * W),
                               lambda c, w: (c // 3, 0, 0))],
        out_specs=[
            pl.BlockSpec((1, RPC, 128), lambda c, w: (w, c, 0)),
            pl.BlockSpec((1, D + 1, VC), lambda c, w: (c // 5, 0, 0)),
        ],
        out_shape=[
            jax.ShapeDtypeStruct((3, ROWS, 128), jnp.int32),
            jax.ShapeDtypeStruct((N, D + 1, VC), jnp.float32),
        ],
        scratch_shapes=[pltpu.VMEM((RPC, 128), jnp.int32),
                        pltpu.VMEM((D + 1, VC), jnp.float32)],
    )(masks)
    faces = faces3.reshape(3, FB, N, 128).transpose(2, 1, 3, 0).reshape(N, NF, 3)
    verts = verts3.reshape(N, (D + 1) * (H + 1) * (W + 1), 3)
    return verts, faces
